# baseline (speedup 1.0000x reference)
import jax
import jax.numpy as jnp
from jax.experimental import pallas as pl
from jax.experimental.pallas import tpu as pltpu

_VMEM_LIMIT = 48 * 1024 * 1024


def _seg_add_kernel(seg_ref, x_ref, tbl_ref, o_ref):
    seg_f = seg_ref[...].astype(jnp.float32)
    tn = seg_f.shape[0]
    s = tbl_ref.shape[0]
    ones_row = jnp.ones((1, s), dtype=jnp.float32)
    seg_b = jnp.dot(seg_f, ones_row,
                    preferred_element_type=jnp.float32)
    ids = jax.lax.broadcasted_iota(jnp.int32, (tn, s), 1).astype(jnp.float32)
    onehot = (ids == seg_b).astype(jnp.bfloat16)
    emb = jnp.dot(onehot, tbl_ref[...],
                  preferred_element_type=jnp.float32)
    o_ref[...] = x_ref[...] + emb


def _pick_tile(n):
    for tn in (2048, 1024, 512, 256, 128, 64, 32, 16, 8):
        if n % tn == 0:
            return tn
    return n


def kernel(x, segment_ids, seg_embed):
    L, B, D = x.shape
    N = L * B
    S = seg_embed.shape[0]
    tn = _pick_tile(N)

    x2d = x.reshape(N, D)
    seg2d = segment_ids.reshape(N, 1).astype(jnp.int32)
    tbl_bf16 = seg_embed.astype(jnp.bfloat16)

    out2d = pl.pallas_call(
        _seg_add_kernel,
        out_shape=jax.ShapeDtypeStruct((N, D), x.dtype),
        grid=(N // tn,),
        in_specs=[
            pl.BlockSpec((tn, 1), lambda i: (i, 0)),
            pl.BlockSpec((tn, D), lambda i: (i, 0)),
            pl.BlockSpec((S, D), lambda i: (0, 0)),
        ],
        out_specs=pl.BlockSpec((tn, D), lambda i: (i, 0)),
        compiler_params=pltpu.CompilerParams(
            dimension_semantics=("parallel",),
            vmem_limit_bytes=_VMEM_LIMIT),
    )(seg2d, x2d, tbl_bf16)
    return out2d.reshape(L, B, D)


# speedup vs baseline: 1.1229x; 1.1229x over previous
import jax
import jax.numpy as jnp
from jax.experimental import pallas as pl
from jax.experimental.pallas import tpu as pltpu

_VMEM_LIMIT = 48 * 1024 * 1024


def _seg_add_kernel(seg_ref, x_ref, tbl_ref, o_ref):
    seg2 = seg_ref[...]
    tn = seg2.shape[0]
    s = tbl_ref.shape[0]
    w = jnp.concatenate(
        [jnp.full((1, s), 256.0, jnp.float32), jnp.ones((1, s), jnp.float32)],
        axis=0)
    seg_b = jnp.dot(seg2, w,
                    preferred_element_type=jnp.float32)
    ids = jax.lax.broadcasted_iota(jnp.int32, (tn, s), 1).astype(jnp.float32)
    onehot = (ids == seg_b).astype(jnp.bfloat16)
    emb = jnp.dot(onehot, tbl_ref[...],
                  preferred_element_type=jnp.float32)
    o_ref[...] = x_ref[...] + emb


def _pick_tile(n):
    for tn in (2048, 1024, 512, 256, 128, 64, 32, 16, 8):
        if n % tn == 0:
            return tn
    return n


def kernel(x, segment_ids, seg_embed):
    L, B, D = x.shape
    N = L * B
    S = seg_embed.shape[0]
    tn = _pick_tile(N)

    x2d = x.reshape(N, D)
    seg = segment_ids.reshape(N).astype(jnp.int32)
    seg2 = jnp.stack([(seg >> 8).astype(jnp.float32),
                      (seg & 255).astype(jnp.float32)], axis=-1)
    tbl_bf16 = seg_embed.astype(jnp.bfloat16)

    out2d = pl.pallas_call(
        _seg_add_kernel,
        out_shape=jax.ShapeDtypeStruct((N, D), x.dtype),
        grid=(N // tn,),
        in_specs=[
            pl.BlockSpec((tn, 2), lambda i: (i, 0)),
            pl.BlockSpec((tn, D), lambda i: (i, 0)),
            pl.BlockSpec((S, D), lambda i: (0, 0)),
        ],
        out_specs=pl.BlockSpec((tn, D), lambda i: (i, 0)),
        compiler_params=pltpu.CompilerParams(
            dimension_semantics=("parallel",),
            vmem_limit_bytes=_VMEM_LIMIT),
    )(seg2, x2d, tbl_bf16)
    return out2d.reshape(L, B, D)


# speedup vs baseline: 1.1455x; 1.0202x over previous
import jax
import jax.numpy as jnp
from jax.experimental import pallas as pl
from jax.experimental.pallas import tpu as pltpu

_VMEM_LIMIT = 48 * 1024 * 1024


def _seg_add_kernel(seg_ref, x_ref, tbl_ref, o_ref):
    seg2 = seg_ref[...]
    tn = seg2.shape[0]
    s = tbl_ref.shape[0]
    w = jnp.concatenate(
        [jnp.full((1, 128), 256.0, jnp.float32), jnp.ones((1, 128), jnp.float32)],
        axis=0)
    seg_b = jnp.dot(seg2, w,
                    preferred_element_type=jnp.float32)
    iota128 = jax.lax.broadcasted_iota(jnp.int32, (tn, 128), 1).astype(jnp.float32)
    groups = [(iota128 + float(g * 128) == seg_b).astype(jnp.bfloat16)
              for g in range(s // 128)]
    onehot = jnp.concatenate(groups, axis=1)
    emb = jnp.dot(onehot, tbl_ref[...],
                  preferred_element_type=jnp.float32)
    o_ref[...] = x_ref[...] + emb


def _pick_tile(n):
    for tn in (2048, 1024, 512, 256, 128, 64, 32, 16, 8):
        if n % tn == 0:
            return tn
    return n


def kernel(x, segment_ids, seg_embed):
    L, B, D = x.shape
    N = L * B
    S = seg_embed.shape[0]
    tn = _pick_tile(N)

    x2d = x.reshape(N, D)
    seg = segment_ids.reshape(N).astype(jnp.int32)
    seg2 = jnp.stack([(seg >> 8).astype(jnp.float32),
                      (seg & 255).astype(jnp.float32)], axis=-1)
    tbl_bf16 = seg_embed.astype(jnp.bfloat16)

    out2d = pl.pallas_call(
        _seg_add_kernel,
        out_shape=jax.ShapeDtypeStruct((N, D), x.dtype),
        grid=(N // tn,),
        in_specs=[
            pl.BlockSpec((tn, 2), lambda i: (i, 0)),
            pl.BlockSpec((tn, D), lambda i: (i, 0)),
            pl.BlockSpec((S, D), lambda i: (0, 0)),
        ],
        out_specs=pl.BlockSpec((tn, D), lambda i: (i, 0)),
        compiler_params=pltpu.CompilerParams(
            dimension_semantics=("parallel",),
            vmem_limit_bytes=_VMEM_LIMIT),
    )(seg2, x2d, tbl_bf16)
    return out2d.reshape(L, B, D)


# speedup vs baseline: 1.1499x; 1.0038x over previous
import jax
import jax.numpy as jnp
from jax.experimental import pallas as pl
from jax.experimental.pallas import tpu as pltpu

_VMEM_LIMIT = 48 * 1024 * 1024


def _seg_add_kernel(seg_ref, x_ref, tbl_ref, o_ref):
    seg2 = seg_ref[...]
    tn = seg2.shape[0]
    s = tbl_ref.shape[0]
    w = jnp.concatenate(
        [jnp.full((1, 128), 256.0, jnp.float32), jnp.ones((1, 128), jnp.float32)],
        axis=0)
    seg_b = jnp.dot(seg2, w,
                    preferred_element_type=jnp.float32)
    iota128 = jax.lax.broadcasted_iota(jnp.int32, (tn, 128), 1).astype(jnp.float32)
    groups = [(iota128 + float(g * 128) == seg_b).astype(jnp.float32)
              for g in range(s // 128)]
    onehot = jnp.concatenate(groups, axis=1)
    emb = jnp.dot(onehot, tbl_ref[...],
                  preferred_element_type=jnp.float32)
    o_ref[...] = x_ref[...] + emb


def _pick_tile(n):
    for tn in (2048, 1024, 512, 256, 128, 64, 32, 16, 8):
        if n % tn == 0:
            return tn
    return n


def kernel(x, segment_ids, seg_embed):
    L, B, D = x.shape
    N = L * B
    S = seg_embed.shape[0]
    tn = _pick_tile(N)

    x2d = x.reshape(N, D)
    seg = segment_ids.reshape(N).astype(jnp.int32)
    seg2 = jnp.stack([(seg >> 8).astype(jnp.float32),
                      (seg & 255).astype(jnp.float32)], axis=-1)
    tbl_in = seg_embed

    out2d = pl.pallas_call(
        _seg_add_kernel,
        out_shape=jax.ShapeDtypeStruct((N, D), x.dtype),
        grid=(N // tn,),
        in_specs=[
            pl.BlockSpec((tn, 2), lambda i: (i, 0)),
            pl.BlockSpec((tn, D), lambda i: (i, 0)),
            pl.BlockSpec((S, D), lambda i: (0, 0)),
        ],
        out_specs=pl.BlockSpec((tn, D), lambda i: (i, 0)),
        compiler_params=pltpu.CompilerParams(
            dimension_semantics=("parallel",),
            vmem_limit_bytes=_VMEM_LIMIT),
    )(seg2, x2d, tbl_in)
    return out2d.reshape(L, B, D)
